# baseline (speedup 1.0000x reference)
import functools

import jax
import jax.numpy as jnp
from jax import lax
from jax.experimental import pallas as pl
from jax.experimental.pallas import tpu as pltpu
from jax.experimental.pallas import tpu_sc as plsc

N_NODES = 10000
D_FEAT = 256
N_GRAPHS = 64
NB = 1024
N_BLOCKS = (N_NODES + NB - 1) // NB
BATCH_PAD = 10240
GB = 96
KMAX = N_NODES // 2 + 1
IDX_CAP = 5168


def _tc_body(x_ref, bat_ref, w_ref, keys_ref, xs_ref, counts_ref, starts_ref):
    xb = x_ref[...]
    w2 = w_ref[...]
    inv_norm = lax.rsqrt(jnp.sum(w2 * w2))
    s = jnp.dot(xb, w2, preferred_element_type=jnp.float32) * inv_norm
    gate = jnp.tanh(s)
    xs_ref[...] = (xb * gate).astype(jnp.bfloat16)
    b = lax.bitcast_convert_type(s, jnp.int32)
    m = lax.shift_right_arithmetic(b, 31)
    keys_ref[...] = (b ^ (m | jnp.int32(-2147483648))).reshape(NB)
    @pl.when(pl.program_id(0) == 0)
    def _meta():
        bat = bat_ref[...].reshape(1, N_NODES)
        gid = lax.broadcasted_iota(jnp.int32, (N_GRAPHS, N_NODES), 0)
        oh = (bat == gid).astype(jnp.float32)
        cf = jnp.sum(oh, axis=1, keepdims=True)
        counts_ref[...] = cf.astype(jnp.int32).reshape(N_GRAPHS)
        r = lax.broadcasted_iota(jnp.int32, (N_GRAPHS, N_GRAPHS), 0)
        c = lax.broadcasted_iota(jnp.int32, (N_GRAPHS, N_GRAPHS), 1)
        tri = (r > c).astype(jnp.float32)
        starts_ref[...] = jnp.dot(
            tri, cf,
            preferred_element_type=jnp.float32).astype(jnp.int32).reshape(N_GRAPHS)


_tc_call = pl.pallas_call(
    _tc_body,
    grid=(N_BLOCKS,),
    in_specs=[
        pl.BlockSpec((NB, D_FEAT), lambda i: (i, 0)),
        pl.BlockSpec((N_NODES,), lambda i: (0,)),
        pl.BlockSpec((D_FEAT, 1), lambda i: (0, 0)),
    ],
    out_specs=[
        pl.BlockSpec((NB,), lambda i: (i,)),
        pl.BlockSpec((NB, D_FEAT), lambda i: (i, 0)),
        pl.BlockSpec((N_GRAPHS,), lambda i: (0,)),
        pl.BlockSpec((N_GRAPHS,), lambda i: (0,)),
    ],
    out_shape=[
        jax.ShapeDtypeStruct((N_NODES,), jnp.int32),
        jax.ShapeDtypeStruct((N_NODES, D_FEAT), jnp.bfloat16),
        jax.ShapeDtypeStruct((N_GRAPHS,), jnp.int32),
        jax.ShapeDtypeStruct((N_GRAPHS,), jnp.int32),
    ],
)


def _iota16():
    return lax.broadcasted_iota(jnp.int32, (16,), 0)


def _sload(ref, i):
    return ref[pl.ds(i, 16)][0]


def _sc_body(keys_hbm, counts_hbm, starts_hbm, xs_hbm, out_hbm,
             keys_v, cnts_v, strt_v, idx_v, rowbuf0_v, rowbuf1_v, accbuf_v,
             sem0, sem1):
    cid = lax.axis_index("c")
    sid = lax.axis_index("s")
    wid = sid * 2 + cid
    pltpu.sync_copy(keys_hbm, keys_v.at[pl.ds(0, N_NODES)])
    pltpu.sync_copy(counts_hbm, cnts_v.at[pl.ds(0, N_GRAPHS)])
    pltpu.sync_copy(starts_hbm, strt_v.at[pl.ds(0, N_GRAPHS)])

    rowbufs = (rowbuf0_v, rowbuf1_v)
    sems = (sem0, sem1)
    ks = []
    nbs = []

    for gi in range(2):
        g = wid * 2 + gi
        n = _sload(cnts_v, g)
        st = _sload(strt_v, g)
        k = (n + 1) // 2
        nch = (n + 15) // 16
        ibase = gi * IDX_CAP

        def _count(thr, strict):
            def ch(c, acc):
                kv = plsc.bitcast(keys_v[pl.ds(st + c * 16, 16)], jnp.uint32)
                valid = _iota16() < (n - c * 16)
                if strict:
                    hit = kv > thr
                else:
                    hit = kv >= thr
                return acc + jnp.where(valid & hit, 1, 0)
            acc = lax.fori_loop(0, nch, ch, jnp.zeros((16,), jnp.int32))
            return jnp.sum(acc)

        def bb(bpos, T):
            bit = lax.shift_right_logical(jnp.uint32(0x80000000),
                                          bpos.astype(jnp.uint32))
            cand = T | bit
            cnt = _count(jnp.broadcast_to(cand, (16,)), strict=False)
            return jnp.where(cnt >= k, cand, T)

        T = lax.fori_loop(0, 32, bb, jnp.uint32(0))
        Tv = jnp.broadcast_to(T, (16,))
        G = _count(Tv, strict=True)
        R = k - G

        def sel_ch(c, carry):
            cnt_sel, ctie = carry
            off = st + c * 16
            kv = plsc.bitcast(keys_v[pl.ds(off, 16)], jnp.uint32)
            valid = _iota16() < (n - c * 16)
            gt = valid & (kv > Tv)
            tie = valid & (kv == Tv)
            tcum = plsc.cumsum(jnp.where(tie, 1, 0))
            sel = gt | (tie & ((tcum + ctie) <= R))
            vals = off + _iota16()
            plsc.store_compressed(idx_v.at[pl.ds(ibase + cnt_sel, 16)],
                                  vals, mask=sel)
            return (cnt_sel + jnp.sum(jnp.where(sel, 1, 0)),
                    ctie + jnp.sum(jnp.where(tie, 1, 0)))

        lax.fori_loop(0, nch, sel_ch, (jnp.int32(0), jnp.int32(0)))

        base = (k // 16) * 16
        for j in range(GB // 16 + 1):
            off = ibase + base + 16 * j
            pos = base + 16 * j + _iota16()
            spread = (g * 311 + pos * 7) % N_NODES
            cur = idx_v[pl.ds(off, 16)]
            idx_v[pl.ds(off, 16)] = jnp.where(pos >= k, spread, cur)

        ks.append(k)
        nbs.append((k + GB - 1) // GB)

    def _slice(gi, b):
        return xs_hbm.at[idx_v.at[pl.ds(gi * IDX_CAP + b * GB, GB)]]

    cps = [pltpu.async_copy(_slice(gi, 0), rowbufs[gi], sems[gi])
           for gi in range(2)]

    for gi in range(2):
        g = wid * 2 + gi
        k = ks[gi]
        nb = nbs[gi]
        rowbuf_v = rowbufs[gi]

        def _acc_batch(b, acc, k=k, rowbuf_v=rowbuf_v):
            rem = jnp.minimum(GB, k - b * GB)

            def rr(r, a):
                new = []
                for t in range(8):
                    u = rowbuf_v[r, pl.ds(t * 16, 16)]
                    fe = plsc.bitcast(u << 16, jnp.float32)
                    fo = plsc.bitcast(u & jnp.int32(-65536), jnp.float32)
                    new.append(a[2 * t] + fe)
                    new.append(a[2 * t + 1] + fo)
                return tuple(new)
            return lax.fori_loop(0, rem, rr, acc)

        acc = tuple(jnp.zeros((16,), jnp.float32) for _ in range(16))
        cps[gi].wait()
        acc = _acc_batch(0, acc)

        def gb_body(b, acc, gi=gi, rowbuf_v=rowbuf_v):
            pltpu.async_copy(_slice(gi, b), rowbuf_v, sems[gi]).wait()
            return _acc_batch(b, acc)

        acc = lax.fori_loop(1, nb, gb_body, acc)

        invk = 1.0 / jnp.broadcast_to(
            jnp.maximum(k, 1).astype(jnp.float32), (16,))
        for t in range(8):
            ev = 32 * t + 2 * _iota16()
            plsc.store_scatter(accbuf_v, [ev], acc[2 * t] * invk)
            plsc.store_scatter(accbuf_v, [ev + 1], acc[2 * t + 1] * invk)
        pltpu.sync_copy(accbuf_v, out_hbm.at[g])


_sc_call = functools.partial(
    pl.kernel,
    out_type=jax.ShapeDtypeStruct((N_GRAPHS, D_FEAT), jnp.float32),
    mesh=plsc.VectorSubcoreMesh(core_axis_name="c", subcore_axis_name="s",
                                num_cores=2, num_subcores=16),
    compiler_params=pltpu.CompilerParams(needs_layout_passes=False),
    scratch_types=[
        pltpu.VMEM((N_NODES + 16,), jnp.int32),
        pltpu.VMEM((N_GRAPHS + 16,), jnp.int32),
        pltpu.VMEM((N_GRAPHS + 16,), jnp.int32),
        pltpu.VMEM((2 * IDX_CAP,), jnp.int32),
        pltpu.VMEM((GB, D_FEAT // 2), jnp.int32),
        pltpu.VMEM((GB, D_FEAT // 2), jnp.int32),
        pltpu.VMEM((D_FEAT,), jnp.float32),
        pltpu.SemaphoreType.DMA,
        pltpu.SemaphoreType.DMA,
    ],
)(_sc_body)


def kernel(x, edge_index, batch, w):
    del edge_index
    keys, xs, counts, starts = _tc_call(x, batch, w.reshape(D_FEAT, 1))
    xs32 = lax.bitcast_convert_type(
        xs.reshape(N_NODES, D_FEAT // 2, 2), jnp.int32)
    return _sc_call(keys, counts, starts, xs32)


# speedup vs baseline: 3.2274x; 3.2274x over previous
import functools

import jax
import jax.numpy as jnp
from jax import lax
from jax.experimental import pallas as pl
from jax.experimental.pallas import tpu as pltpu
from jax.experimental.pallas import tpu_sc as plsc

N_NODES = 10000
D_FEAT = 256
N_GRAPHS = 64
NB = 1024
N_BLOCKS = (N_NODES + NB - 1) // NB
BATCH_PAD = 10240
GB = 96
KMAX = N_NODES // 2 + 1
IDX_CAP = 5168


def _tc_body(x_ref, bat_ref, w_ref, keys_ref, xs_ref, counts_ref, starts_ref):
    xb = x_ref[...]
    w2 = w_ref[...]
    inv_norm = lax.rsqrt(jnp.sum(w2 * w2))
    s = jnp.dot(xb, w2, preferred_element_type=jnp.float32) * inv_norm
    gate = jnp.tanh(s)
    xsf = xb * gate
    au = lax.bitcast_convert_type(xsf[:, :D_FEAT // 2], jnp.int32)
    bu = lax.bitcast_convert_type(xsf[:, D_FEAT // 2:], jnp.int32)
    au = au + 0x7FFF + (lax.shift_right_logical(au, 16) & 1)
    bu = bu + 0x7FFF + (lax.shift_right_logical(bu, 16) & 1)
    xs_ref[...] = (lax.shift_right_logical(au, 16)
                   | (bu & jnp.int32(-65536)))
    b = lax.bitcast_convert_type(s, jnp.int32)
    m = lax.shift_right_arithmetic(b, 31)
    keys_ref[...] = (b ^ (m | jnp.int32(-2147483648))).reshape(NB)
    @pl.when(pl.program_id(0) == 0)
    def _meta():
        bat = bat_ref[...].reshape(1, N_NODES)
        gid = lax.broadcasted_iota(jnp.int32, (N_GRAPHS, N_NODES), 0)
        oh = (bat == gid).astype(jnp.float32)
        cf = jnp.sum(oh, axis=1, keepdims=True)
        counts_ref[...] = cf.astype(jnp.int32).reshape(N_GRAPHS)
        r = lax.broadcasted_iota(jnp.int32, (N_GRAPHS, N_GRAPHS), 0)
        c = lax.broadcasted_iota(jnp.int32, (N_GRAPHS, N_GRAPHS), 1)
        tri = (r > c).astype(jnp.float32)
        starts_ref[...] = jnp.dot(
            tri, cf,
            preferred_element_type=jnp.float32).astype(jnp.int32).reshape(N_GRAPHS)


_tc_call = pl.pallas_call(
    _tc_body,
    grid=(N_BLOCKS,),
    in_specs=[
        pl.BlockSpec((NB, D_FEAT), lambda i: (i, 0)),
        pl.BlockSpec((N_NODES,), lambda i: (0,)),
        pl.BlockSpec((D_FEAT, 1), lambda i: (0, 0)),
    ],
    out_specs=[
        pl.BlockSpec((NB,), lambda i: (i,)),
        pl.BlockSpec((NB, D_FEAT // 2), lambda i: (i, 0)),
        pl.BlockSpec((N_GRAPHS,), lambda i: (0,)),
        pl.BlockSpec((N_GRAPHS,), lambda i: (0,)),
    ],
    out_shape=[
        jax.ShapeDtypeStruct((N_NODES,), jnp.int32),
        jax.ShapeDtypeStruct((N_NODES, D_FEAT // 2), jnp.int32),
        jax.ShapeDtypeStruct((N_GRAPHS,), jnp.int32),
        jax.ShapeDtypeStruct((N_GRAPHS,), jnp.int32),
    ],
)


def _iota16():
    return lax.broadcasted_iota(jnp.int32, (16,), 0)


def _sload(ref, i):
    return ref[pl.ds(i, 16)][0]


def _sc_body(keys_hbm, counts_hbm, starts_hbm, xs_hbm, out_hbm,
             keys_v, cnts_v, strt_v, idx_v, rowbuf0_v, rowbuf1_v, accbuf_v,
             sem0, sem1):
    cid = lax.axis_index("c")
    sid = lax.axis_index("s")
    wid = sid * 2 + cid
    pltpu.sync_copy(keys_hbm, keys_v.at[pl.ds(0, N_NODES)])
    pltpu.sync_copy(counts_hbm, cnts_v.at[pl.ds(0, N_GRAPHS)])
    pltpu.sync_copy(starts_hbm, strt_v.at[pl.ds(0, N_GRAPHS)])

    rowbufs = (rowbuf0_v, rowbuf1_v)
    sems = (sem0, sem1)
    ks = []
    nbs = []

    for gi in range(2):
        g = wid * 2 + gi
        n = _sload(cnts_v, g)
        st = _sload(strt_v, g)
        k = (n + 1) // 2
        nch = (n + 15) // 16
        ibase = gi * IDX_CAP

        def _count(thr, strict):
            def ch(c, acc):
                kv = plsc.bitcast(keys_v[pl.ds(st + c * 16, 16)], jnp.uint32)
                valid = _iota16() < (n - c * 16)
                if strict:
                    hit = kv > thr
                else:
                    hit = kv >= thr
                return acc + jnp.where(valid & hit, 1, 0)
            acc = lax.fori_loop(0, nch, ch, jnp.zeros((16,), jnp.int32))
            return jnp.sum(acc)

        def bb(bpos, T):
            bit = lax.shift_right_logical(jnp.uint32(0x80000000),
                                          bpos.astype(jnp.uint32))
            cand = T | bit
            cnt = _count(jnp.broadcast_to(cand, (16,)), strict=False)
            return jnp.where(cnt >= k, cand, T)

        T = lax.fori_loop(0, 32, bb, jnp.uint32(0))
        Tv = jnp.broadcast_to(T, (16,))
        G = _count(Tv, strict=True)
        R = k - G

        def sel_ch(c, carry):
            cnt_sel, ctie = carry
            off = st + c * 16
            kv = plsc.bitcast(keys_v[pl.ds(off, 16)], jnp.uint32)
            valid = _iota16() < (n - c * 16)
            gt = valid & (kv > Tv)
            tie = valid & (kv == Tv)
            tcum = plsc.cumsum(jnp.where(tie, 1, 0))
            sel = gt | (tie & ((tcum + ctie) <= R))
            vals = off + _iota16()
            plsc.store_compressed(idx_v.at[pl.ds(ibase + cnt_sel, 16)],
                                  vals, mask=sel)
            return (cnt_sel + jnp.sum(jnp.where(sel, 1, 0)),
                    ctie + jnp.sum(jnp.where(tie, 1, 0)))

        lax.fori_loop(0, nch, sel_ch, (jnp.int32(0), jnp.int32(0)))

        base = (k // 16) * 16
        for j in range(GB // 16 + 1):
            off = ibase + base + 16 * j
            pos = base + 16 * j + _iota16()
            spread = (g * 311 + pos * 7) % N_NODES
            cur = idx_v[pl.ds(off, 16)]
            idx_v[pl.ds(off, 16)] = jnp.where(pos >= k, spread, cur)

        ks.append(k)
        nbs.append((k + GB - 1) // GB)

    def _slice(gi, b):
        return xs_hbm.at[idx_v.at[pl.ds(gi * IDX_CAP + b * GB, GB)]]

    cps = [pltpu.async_copy(_slice(gi, 0), rowbufs[gi], sems[gi])
           for gi in range(2)]

    for gi in range(2):
        g = wid * 2 + gi
        k = ks[gi]
        nb = nbs[gi]
        rowbuf_v = rowbufs[gi]

        def _acc_batch(b, acc, k=k, rowbuf_v=rowbuf_v):
            rem = jnp.minimum(GB, k - b * GB)

            def rr(r, a):
                new = [None] * 16
                for t in range(8):
                    u = rowbuf_v[r, pl.ds(t * 16, 16)]
                    flo = plsc.bitcast(u << 16, jnp.float32)
                    fhi = plsc.bitcast(u & jnp.int32(-65536), jnp.float32)
                    new[t] = a[t] + flo
                    new[8 + t] = a[8 + t] + fhi
                return tuple(new)
            return lax.fori_loop(0, rem, rr, acc)

        acc = tuple(jnp.zeros((16,), jnp.float32) for _ in range(16))
        cps[gi].wait()
        acc = _acc_batch(0, acc)

        def gb_body(b, acc, gi=gi, rowbuf_v=rowbuf_v):
            pltpu.async_copy(_slice(gi, b), rowbuf_v, sems[gi]).wait()
            return _acc_batch(b, acc)

        acc = lax.fori_loop(1, nb, gb_body, acc)

        invk = 1.0 / jnp.broadcast_to(
            jnp.maximum(k, 1).astype(jnp.float32), (16,))
        for t in range(8):
            accbuf_v[pl.ds(16 * t, 16)] = acc[t] * invk
            accbuf_v[pl.ds(128 + 16 * t, 16)] = acc[8 + t] * invk
        pltpu.sync_copy(accbuf_v, out_hbm.at[g])


_sc_call = functools.partial(
    pl.kernel,
    out_type=jax.ShapeDtypeStruct((N_GRAPHS, D_FEAT), jnp.float32),
    mesh=plsc.VectorSubcoreMesh(core_axis_name="c", subcore_axis_name="s",
                                num_cores=2, num_subcores=16),
    compiler_params=pltpu.CompilerParams(needs_layout_passes=False),
    scratch_types=[
        pltpu.VMEM((N_NODES + 16,), jnp.int32),
        pltpu.VMEM((N_GRAPHS + 16,), jnp.int32),
        pltpu.VMEM((N_GRAPHS + 16,), jnp.int32),
        pltpu.VMEM((2 * IDX_CAP,), jnp.int32),
        pltpu.VMEM((GB, D_FEAT // 2), jnp.int32),
        pltpu.VMEM((GB, D_FEAT // 2), jnp.int32),
        pltpu.VMEM((D_FEAT,), jnp.float32),
        pltpu.SemaphoreType.DMA,
        pltpu.SemaphoreType.DMA,
    ],
)(_sc_body)


def kernel(x, edge_index, batch, w):
    del edge_index
    keys, xs32, counts, starts = _tc_call(x, batch, w.reshape(D_FEAT, 1))
    return _sc_call(keys, counts, starts, xs32)
